# baseline (speedup 1.0000x reference)
import functools

import jax
import jax.numpy as jnp
from jax import lax
from jax.experimental import pallas as pl
from jax.experimental.pallas import tpu as pltpu
from jax.experimental.pallas import tpu_sc as plsc

_STRIDE = 64
_NSEL = 64
_NUM_CORES = 2
_NUM_SUBCORES = 16
_NW = _NUM_CORES * _NUM_SUBCORES
_LANES = 16
_CHUNK = 4096
_INNER = 8


def kernel(x):
    b, s, f = x.shape
    assert f == _STRIDE * _NSEL
    total = b * s * _NSEL
    per_w = total // _NW
    nchunks = per_w // _CHUNK
    rows = b * s
    xp = (
        x.reshape(rows // 8, 8, f // 128, 128)
        .transpose(0, 2, 1, 3)
        .reshape(total, _STRIDE)
    )

    mesh = plsc.VectorSubcoreMesh(core_axis_name="c", subcore_axis_name="s")

    @functools.partial(
        pl.kernel,
        mesh=mesh,
        out_type=jax.ShapeDtypeStruct((2 * total,), jnp.float32),
        scratch_types=[
            pltpu.VMEM((3, _CHUNK, _INNER), jnp.float32),
            pltpu.VMEM((2, 2 * _CHUNK), jnp.float32),
            pltpu.SemaphoreType.DMA,
            pltpu.SemaphoreType.DMA,
            pltpu.SemaphoreType.DMA,
            pltpu.SemaphoreType.DMA,
            pltpu.SemaphoreType.DMA,
            pltpu.SemaphoreType.DMA,
            pltpu.SemaphoreType.DMA,
            pltpu.SemaphoreType.DMA,
        ],
        compiler_params=pltpu.CompilerParams(
            use_tc_tiling_on_sc=False, needs_layout_passes=False
        ),
    )
    def run(
        x_hbm,
        out_hbm,
        inbuf,
        outbuf,
        in_sem0a,
        in_sem0b,
        in_sem1a,
        in_sem1b,
        in_sem2a,
        in_sem2b,
        out_sem0,
        out_sem1,
    ):
        wid = lax.axis_index("s") * _NUM_CORES + lax.axis_index("c")
        base = wid * per_w
        in_sems = (
            (in_sem0a, in_sem0b),
            (in_sem1a, in_sem1b),
            (in_sem2a, in_sem2b),
        )
        out_sems = (out_sem0, out_sem1)
        quarter = _CHUNK // 4
        lane = lax.iota(jnp.int32, _LANES)
        svr = jnp.left_shift(jnp.right_shift(lane, 1), 4) + jnp.bitwise_and(
            lane, 1
        )
        zeros = jnp.zeros((_LANES,), jnp.int32)

        def fetch(c, slot):
            r0 = base + c * _CHUNK
            return [
                pltpu.async_copy(
                    x_hbm.at[pl.ds(r0 + h * quarter, quarter), pl.ds(0, _INNER)],
                    inbuf.at[slot, pl.ds(h * quarter, quarter)],
                    in_sems[slot][h % 2],
                )
                for h in range(4)
            ]

        in_pending = [fetch(0, 0), fetch(1, 1)]
        out_pending = [None, None]
        for c in range(nchunks):
            slot = c % 3
            oslot = c % 2
            for h in in_pending.pop(0):
                h.wait()
            if c + 2 < nchunks:
                in_pending.append(fetch(c + 2, (c + 2) % 3))
            if out_pending[oslot] is not None:
                out_pending[oslot].wait()

            def compact(i, _):
                sbase = (
                    jnp.left_shift(jnp.right_shift(i, 5), 9)
                    + jnp.left_shift(jnp.bitwise_and(i, 3), 7)
                    + jnp.left_shift(jnp.bitwise_and(jnp.right_shift(i, 2), 7), 1)
                )
                src = sbase + svr
                vals = plsc.load_gather(inbuf.at[slot], [src, zeros])
                pbase = jnp.left_shift(jnp.right_shift(i, 2), 7) + jnp.left_shift(
                    jnp.bitwise_and(i, 3), 4
                )
                outbuf[oslot, pl.ds(pbase, _LANES)] = vals
                return _

            lax.fori_loop(0, _CHUNK // _LANES, compact, None, unroll=2)
            out_pending[oslot] = pltpu.async_copy(
                outbuf.at[oslot],
                out_hbm.at[pl.ds(2 * (base + c * _CHUNK), 2 * _CHUNK)],
                out_sems[oslot],
            )
        for h in out_pending:
            if h is not None:
                h.wait()

    return run(xp).reshape(rows, 2 * _NSEL)[:, :_NSEL].reshape(b, s, _NSEL)


# speedup vs baseline: 1.0076x; 1.0076x over previous
import functools

import jax
import jax.numpy as jnp
from jax import lax
from jax.experimental import pallas as pl
from jax.experimental.pallas import tpu as pltpu
from jax.experimental.pallas import tpu_sc as plsc

_STRIDE = 64
_NSEL = 64
_NUM_CORES = 2
_NUM_SUBCORES = 16
_NW = _NUM_CORES * _NUM_SUBCORES
_LANES = 16
_CHUNK = 4096
_INNER = 8


def kernel(x):
    b, s, f = x.shape
    assert f == _STRIDE * _NSEL
    total = b * s * _NSEL
    per_w = total // _NW
    nchunks = per_w // _CHUNK
    rows = b * s
    xp = (
        x.reshape(rows // 8, 8, f // 128, 128)
        .transpose(0, 2, 1, 3)
        .reshape(total, _STRIDE)
    )

    mesh = plsc.VectorSubcoreMesh(core_axis_name="c", subcore_axis_name="s")

    @functools.partial(
        pl.kernel,
        mesh=mesh,
        out_type=jax.ShapeDtypeStruct((2 * total,), jnp.float32),
        scratch_types=[
            pltpu.VMEM((3, _CHUNK, _INNER), jnp.float32),
            pltpu.VMEM((2, 2 * _CHUNK), jnp.float32),
            pltpu.SemaphoreType.DMA,
            pltpu.SemaphoreType.DMA,
            pltpu.SemaphoreType.DMA,
            pltpu.SemaphoreType.DMA,
            pltpu.SemaphoreType.DMA,
            pltpu.SemaphoreType.DMA,
            pltpu.SemaphoreType.DMA,
            pltpu.SemaphoreType.DMA,
        ],
        compiler_params=pltpu.CompilerParams(
            use_tc_tiling_on_sc=False, needs_layout_passes=False
        ),
    )
    def run(
        x_hbm,
        out_hbm,
        inbuf,
        outbuf,
        in_sem0a,
        in_sem0b,
        in_sem1a,
        in_sem1b,
        in_sem2a,
        in_sem2b,
        out_sem0,
        out_sem1,
    ):
        wid = lax.axis_index("s") * _NUM_CORES + lax.axis_index("c")
        base = wid * per_w
        in_sems = (
            (in_sem0a, in_sem0b),
            (in_sem1a, in_sem1b),
            (in_sem2a, in_sem2b),
        )
        out_sems = (out_sem0, out_sem1)
        half = _CHUNK // 2
        lane = lax.iota(jnp.int32, _LANES)
        svr = jnp.left_shift(jnp.right_shift(lane, 1), 4) + jnp.bitwise_and(
            lane, 1
        )
        zeros = jnp.zeros((_LANES,), jnp.int32)

        def fetch(c, slot):
            r0 = base + c * _CHUNK
            return [
                pltpu.async_copy(
                    x_hbm.at[pl.ds(r0 + h * half, half), pl.ds(0, _INNER)],
                    inbuf.at[slot, pl.ds(h * half, half)],
                    in_sems[slot][h],
                )
                for h in range(2)
            ]

        in_pending = [fetch(0, 0), fetch(1, 1)]
        out_pending = [None, None]
        for c in range(nchunks):
            slot = c % 3
            oslot = c % 2
            for h in in_pending.pop(0):
                h.wait()
            if c + 2 < nchunks:
                in_pending.append(fetch(c + 2, (c + 2) % 3))
            if out_pending[oslot] is not None:
                out_pending[oslot].wait()

            def compact(i, _):
                sbase = (
                    jnp.left_shift(jnp.right_shift(i, 5), 9)
                    + jnp.left_shift(jnp.bitwise_and(i, 3), 7)
                    + jnp.left_shift(jnp.bitwise_and(jnp.right_shift(i, 2), 7), 1)
                )
                src = sbase + svr
                vals = plsc.load_gather(inbuf.at[slot], [src, zeros])
                pbase = jnp.left_shift(jnp.right_shift(i, 2), 7) + jnp.left_shift(
                    jnp.bitwise_and(i, 3), 4
                )
                outbuf[oslot, pl.ds(pbase, _LANES)] = vals
                return _

            lax.fori_loop(0, _CHUNK // _LANES, compact, None, unroll=2)
            out_pending[oslot] = pltpu.async_copy(
                outbuf.at[oslot],
                out_hbm.at[pl.ds(2 * (base + c * _CHUNK), 2 * _CHUNK)],
                out_sems[oslot],
            )
        for h in out_pending:
            if h is not None:
                h.wait()

    return run(xp).reshape(rows, 2 * _NSEL)[:, :_NSEL].reshape(b, s, _NSEL)
